# baseline (speedup 1.0000x reference)
import functools

import jax
import jax.numpy as jnp
from jax import lax
from jax.experimental import pallas as pl
from jax.experimental.pallas import tpu as pltpu
from jax.experimental.pallas import tpu_sc as plsc

B, P, D = 64, 1024, 768
NW = 32
RPW = P // NW
LANES = 16
NCH = 2
CR = RPW // NCH


@functools.partial(
    pl.kernel,
    mesh=plsc.VectorSubcoreMesh(core_axis_name="c", subcore_axis_name="s"),
    out_type=jax.ShapeDtypeStruct((P, D), jnp.float32),
    scratch_types=(
        [pltpu.VMEM((RPW,), jnp.int32)]
        + [pltpu.VMEM((CR, D), jnp.float32) for _ in range(NCH)]
        + [pltpu.SemaphoreType.DMA for _ in range(NCH)]
    ),
)
def _sc_lookup(table_hbm, out_hbm, idx_v, *refs):
    rows = list(refs[:NCH])
    sems = list(refs[NCH:])
    w = lax.axis_index("s") * 2 + lax.axis_index("c")
    base = w * RPW
    for j in range(RPW // LANES):
        idx_v[pl.ds(j * LANES, LANES)] = (
            base + j * LANES + lax.iota(jnp.int32, LANES))
    for ch in range(NCH):
        pltpu.async_copy(table_hbm.at[idx_v.at[pl.ds(ch * CR, CR)]],
                         rows[ch], sems[ch])
    for ch in range(NCH):
        pltpu.make_async_copy(table_hbm.at[idx_v.at[pl.ds(ch * CR, CR)]],
                              rows[ch], sems[ch]).wait()
        pltpu.sync_copy(rows[ch], out_hbm.at[pl.ds(base + ch * CR, CR)])


def _add_body(patch_ref, pos_ref, out_ref):
    out_ref[...] = patch_ref[...] + pos_ref[...]


TC_BB = 4


def _tc_add(patch, pos):
    return pl.pallas_call(
        _add_body,
        grid=(B // TC_BB,),
        in_specs=[
            pl.BlockSpec((TC_BB, P, D), lambda b: (b, 0, 0)),
            pl.BlockSpec((P, D), lambda b: (0, 0)),
        ],
        out_specs=pl.BlockSpec((TC_BB, P, D), lambda b: (b, 0, 0)),
        out_shape=jax.ShapeDtypeStruct((B, P, D), patch.dtype),
    )(patch, pos)


def kernel(patch, pos_emb_table):
    gathered = _sc_lookup(pos_emb_table)
    return _tc_add(patch, gathered)
